# baseline (speedup 1.0000x reference)
import functools

import jax
import jax.numpy as jnp
from jax import lax
from jax.experimental import pallas as pl
from jax.experimental.pallas import tpu as pltpu
from jax.experimental.pallas import tpu_sc as plsc

B, T, D = 64, 576, 64
K = 1024
TOK = B * T
DP = 128

BB = 2
BT = BB * T
G = TOK // BT
KC = 128
NKC = K // KC
RT = BT // 128

NC, NS = 2, 16
NW = NC * NS
CH = 128
NCHT = TOK // CH
GH = G // 2
NCHP = NCHT // 2
NT = -(-NCHP // NW)
NBUF = 3


def _argmin_body(x_ref, cb_ref, idx_ref, cbp_ref):
    halves = []
    for h in range(BB):
        xh = x_ref[h]
        best = jnp.full((1, T), jnp.inf, jnp.float32)
        besti = jnp.zeros((1, T), jnp.int32)
        iota = lax.broadcasted_iota(jnp.int32, (KC, T), 0)
        for c in range(NKC):
            cb_c = cb_ref[pl.ds(c * KC, KC), :]
            mm = lax.dot_general(cb_c, xh, (((1,), (0,)), ((), ())),
                                 preferred_element_type=jnp.float32)
            cbn = jnp.sum(cb_c * cb_c, axis=1)
            d = cbn[:, None] * 0.5 - mm
            m = jnp.min(d, axis=0, keepdims=True)
            i = jnp.min(jnp.where(d == m, iota, KC),
                        axis=0, keepdims=True)
            take = m < best
            besti = jnp.where(take, i + c * KC, besti)
            best = jnp.minimum(best, m)
        halves.append(besti)
    idx_ref[...] = jnp.concatenate(halves, axis=1).reshape(1, RT, CH)

    @pl.when(pl.program_id(0) == 0)
    def _():
        cbp_ref[...] = jnp.concatenate(
            [cb_ref[...], jnp.zeros((K, DP - D), jnp.float32)], axis=1)


def _make_tc(off):
    return pl.pallas_call(
        _argmin_body,
        grid=(GH,),
        in_specs=[
            pl.BlockSpec((BB, D, T), lambda i: (i + off, 0, 0)),
            pl.BlockSpec((K, D), lambda i: (0, 0)),
        ],
        out_specs=[
            pl.BlockSpec((1, RT, CH), lambda i: (i, 0, 0)),
            pl.BlockSpec((K, DP), lambda i: (0, 0)),
        ],
        out_shape=[
            jax.ShapeDtypeStruct((GH, RT, CH), jnp.int32),
            jax.ShapeDtypeStruct((K, DP), jnp.float32),
        ],
    )


_tc_a = _make_tc(0)
_tc_b = _make_tc(GH)


@functools.cache
def _make_sc_gather():
    @functools.partial(
        pl.kernel,
        out_type=jax.ShapeDtypeStruct((TOK // 2, DP), jnp.float32),
        mesh=plsc.VectorSubcoreMesh(core_axis_name="c", subcore_axis_name="s"),
        scratch_types=[
            pltpu.VMEM((NT, CH), jnp.int32),
            pltpu.VMEM((NBUF, CH, DP), jnp.float32),
            pltpu.SemaphoreType.DMA,
            pltpu.SemaphoreType.DMA,
            pltpu.SemaphoreType.DMA,
        ],
    )
    def _sc_gather(table_hbm, idx_hbm, out_hbm, idx_v, buf_v,
                   sem_i, sem_g, sem_w):
        wid = lax.axis_index("s") * NC + lax.axis_index("c")
        ms = [jnp.minimum(wid + NW * t, NCHP - 1) for t in range(NT)]
        icp = [None] * NT
        gcp = [None] * NT
        wcp = [None] * NT
        for t in range(NT):
            icp[t] = pltpu.async_copy(idx_hbm.at[ms[t] // RT, ms[t] % RT],
                                      idx_v.at[t], sem_i)
        for t in range(NT):
            if t >= NBUF:
                wcp[t - NBUF].wait()
            icp[t].wait()
            gcp[t] = pltpu.async_copy(table_hbm.at[idx_v.at[t]],
                                      buf_v.at[t % NBUF], sem_g)
            if t >= 1:
                gcp[t - 1].wait()
                wcp[t - 1] = pltpu.async_copy(
                    buf_v.at[(t - 1) % NBUF],
                    out_hbm.at[pl.ds(ms[t - 1] * CH, CH)], sem_w)
        gcp[NT - 1].wait()
        wcp[NT - 1] = pltpu.async_copy(
            buf_v.at[(NT - 1) % NBUF],
            out_hbm.at[pl.ds(ms[NT - 1] * CH, CH)], sem_w)
        for t in range(NT - NBUF, NT):
            wcp[t].wait()

    return _sc_gather


def kernel(x, codebook):
    xt = jnp.swapaxes(x, 1, 2)
    idx_a, cbp = _tc_a(xt, codebook)
    q_a = _make_sc_gather()(cbp, idx_a)
    idx_b, _ = _tc_b(xt, codebook)
    q_b = _make_sc_gather()(cbp, idx_b)
    q = jnp.concatenate([q_a, q_b], axis=0)
    return q[:, :D].reshape(B, T, D)


# speedup vs baseline: 1.2237x; 1.2237x over previous
import functools

import jax
import jax.numpy as jnp
from jax import lax
from jax.experimental import pallas as pl
from jax.experimental.pallas import tpu as pltpu
from jax.experimental.pallas import tpu_sc as plsc

B, T, D = 64, 576, 64
K = 1024
TOK = B * T
DP = 128

BB = 2
BT = BB * T
G = TOK // BT
KC = 128
NKC = K // KC
RT = BT // 128

NC, NS = 2, 16
NW = NC * NS
CH = 128
NCHT = TOK // CH
NCH = NCHT // NW
NBUF = 3


def _argmin_body(x_ref, cb_ref, idx_ref, cbp_ref):
    halves = []
    for h in range(BB):
        xh = x_ref[h]
        best = jnp.full((1, T), jnp.inf, jnp.float32)
        besti = jnp.zeros((1, T), jnp.int32)
        iota = lax.broadcasted_iota(jnp.int32, (KC, T), 0)
        for c in range(NKC):
            cb_c = cb_ref[pl.ds(c * KC, KC), :]
            mm = lax.dot_general(cb_c, xh, (((1,), (0,)), ((), ())),
                                 preferred_element_type=jnp.float32)
            cbn = jnp.sum(cb_c * cb_c, axis=1)
            d = cbn[:, None] * 0.5 - mm
            m = jnp.min(d, axis=0, keepdims=True)
            i = jnp.min(jnp.where(d == m, iota, KC),
                        axis=0, keepdims=True)
            take = m < best
            besti = jnp.where(take, i + c * KC, besti)
            best = jnp.minimum(best, m)
        halves.append(besti)
    idx_ref[...] = jnp.concatenate(halves, axis=1).reshape(1, RT, CH)

    @pl.when(pl.program_id(0) == 0)
    def _():
        cbp_ref[...] = jnp.concatenate(
            [cb_ref[...], jnp.zeros((K, DP - D), jnp.float32)], axis=1)


_tc_argmin = pl.pallas_call(
    _argmin_body,
    grid=(G,),
    in_specs=[
        pl.BlockSpec((BB, D, T), lambda i: (i, 0, 0)),
        pl.BlockSpec((K, D), lambda i: (0, 0)),
    ],
    out_specs=[
        pl.BlockSpec((1, RT, CH), lambda i: (i, 0, 0)),
        pl.BlockSpec((K, DP), lambda i: (0, 0)),
    ],
    out_shape=[
        jax.ShapeDtypeStruct((G, RT, CH), jnp.int32),
        jax.ShapeDtypeStruct((K, DP), jnp.float32),
    ],
)


@functools.cache
def _make_sc_gather():
    @functools.partial(
        pl.kernel,
        out_type=jax.ShapeDtypeStruct((TOK, DP), jnp.float32),
        mesh=plsc.VectorSubcoreMesh(core_axis_name="c", subcore_axis_name="s"),
        scratch_types=[
            pltpu.VMEM((NCH, CH), jnp.int32),
            pltpu.VMEM((NBUF, CH, DP), jnp.float32),
            pltpu.SemaphoreType.DMA,
            pltpu.SemaphoreType.DMA,
            pltpu.SemaphoreType.DMA,
        ],
    )
    def _sc_gather(table_hbm, idx_hbm, out_hbm, idx_v, buf_v,
                   sem_i, sem_g, sem_w):
        wid = lax.axis_index("s") * NC + lax.axis_index("c")
        icp = [None] * NCH
        for t in range(NCH):
            m = wid + NW * t
            icp[t] = pltpu.async_copy(idx_hbm.at[m // RT, m % RT],
                                      idx_v.at[t], sem_i)
        gcp = [None] * NCH
        wcp = [None] * NCH
        for t in range(NCH):
            if t >= NBUF:
                wcp[t - NBUF].wait()
            icp[t].wait()
            gcp[t] = pltpu.async_copy(table_hbm.at[idx_v.at[t]],
                                      buf_v.at[t % NBUF], sem_g)
            if t >= 1:
                gcp[t - 1].wait()
                m = wid + NW * (t - 1)
                wcp[t - 1] = pltpu.async_copy(
                    buf_v.at[(t - 1) % NBUF],
                    out_hbm.at[pl.ds(m * CH, CH)], sem_w)
        gcp[NCH - 1].wait()
        m_last = wid + NW * (NCH - 1)
        wcp[NCH - 1] = pltpu.async_copy(
            buf_v.at[(NCH - 1) % NBUF],
            out_hbm.at[pl.ds(m_last * CH, CH)], sem_w)
        for t in range(NCH - NBUF, NCH):
            wcp[t].wait()

    return _sc_gather


def kernel(x, codebook):
    xt = jnp.swapaxes(x, 1, 2)
    idx, cbp = _tc_argmin(xt, codebook)
    q = _make_sc_gather()(cbp, idx)
    return q[:, :D].reshape(B, T, D)


# speedup vs baseline: 1.2399x; 1.0132x over previous
import functools

import jax
import jax.numpy as jnp
from jax import lax
from jax.experimental import pallas as pl
from jax.experimental.pallas import tpu as pltpu
from jax.experimental.pallas import tpu_sc as plsc

B, T, D = 64, 576, 64
K = 1024
TOK = B * T
DP = 128

BB = 2
BT = BB * T
G = TOK // BT
KC = 256
NKC = K // KC
RT = BT // 128

NC, NS = 2, 16
NW = NC * NS
CH = 128
NCHT = TOK // CH
NCH = NCHT // NW
NBUF = 3


def _argmin_body(x_ref, cb_ref, idx_ref, cbp_ref):
    halves = []
    for h in range(BB):
        xh = x_ref[h]
        best = jnp.full((1, T), jnp.inf, jnp.float32)
        besti = jnp.zeros((1, T), jnp.int32)
        iota = lax.broadcasted_iota(jnp.int32, (KC, T), 0)
        for c in range(NKC):
            cb_c = cb_ref[pl.ds(c * KC, KC), :]
            mm = lax.dot_general(cb_c, xh, (((1,), (0,)), ((), ())),
                                 preferred_element_type=jnp.float32)
            cbn = jnp.sum(cb_c * cb_c, axis=1)
            d = cbn[:, None] * 0.5 - mm
            m = jnp.min(d, axis=0, keepdims=True)
            i = jnp.min(jnp.where(d == m, iota, KC),
                        axis=0, keepdims=True)
            take = m < best
            besti = jnp.where(take, i + c * KC, besti)
            best = jnp.minimum(best, m)
        halves.append(besti)
    idx_ref[...] = jnp.concatenate(halves, axis=1).reshape(1, RT, CH)

    @pl.when(pl.program_id(0) == 0)
    def _():
        cbp_ref[...] = jnp.concatenate(
            [cb_ref[...], jnp.zeros((K, DP - D), jnp.float32)], axis=1)


_tc_argmin = pl.pallas_call(
    _argmin_body,
    grid=(G,),
    in_specs=[
        pl.BlockSpec((BB, D, T), lambda i: (i, 0, 0)),
        pl.BlockSpec((K, D), lambda i: (0, 0)),
    ],
    out_specs=[
        pl.BlockSpec((1, RT, CH), lambda i: (i, 0, 0)),
        pl.BlockSpec((K, DP), lambda i: (0, 0)),
    ],
    out_shape=[
        jax.ShapeDtypeStruct((G, RT, CH), jnp.int32),
        jax.ShapeDtypeStruct((K, DP), jnp.float32),
    ],
)


@functools.cache
def _make_sc_gather():
    @functools.partial(
        pl.kernel,
        out_type=jax.ShapeDtypeStruct((TOK, DP), jnp.float32),
        mesh=plsc.VectorSubcoreMesh(core_axis_name="c", subcore_axis_name="s"),
        scratch_types=[
            pltpu.VMEM((NCH, CH), jnp.int32),
            pltpu.VMEM((NBUF, CH, DP), jnp.float32),
            pltpu.SemaphoreType.DMA,
            pltpu.SemaphoreType.DMA,
            pltpu.SemaphoreType.DMA,
        ],
    )
    def _sc_gather(table_hbm, idx_hbm, out_hbm, idx_v, buf_v,
                   sem_i, sem_g, sem_w):
        wid = lax.axis_index("s") * NC + lax.axis_index("c")
        icp = [None] * NCH
        for t in range(NCH):
            m = wid + NW * t
            icp[t] = pltpu.async_copy(idx_hbm.at[m // RT, m % RT],
                                      idx_v.at[t], sem_i)
        gcp = [None] * NCH
        wcp = [None] * NCH
        for t in range(NCH):
            if t >= NBUF:
                wcp[t - NBUF].wait()
            icp[t].wait()
            gcp[t] = pltpu.async_copy(table_hbm.at[idx_v.at[t]],
                                      buf_v.at[t % NBUF], sem_g)
            if t >= 1:
                gcp[t - 1].wait()
                m = wid + NW * (t - 1)
                wcp[t - 1] = pltpu.async_copy(
                    buf_v.at[(t - 1) % NBUF],
                    out_hbm.at[pl.ds(m * CH, CH)], sem_w)
        gcp[NCH - 1].wait()
        m_last = wid + NW * (NCH - 1)
        wcp[NCH - 1] = pltpu.async_copy(
            buf_v.at[(NCH - 1) % NBUF],
            out_hbm.at[pl.ds(m_last * CH, CH)], sem_w)
        for t in range(NCH - NBUF, NCH):
            wcp[t].wait()

    return _sc_gather


def kernel(x, codebook):
    xt = jnp.swapaxes(x, 1, 2)
    idx, cbp = _tc_argmin(xt, codebook)
    q = _make_sc_gather()(cbp, idx)
    return q[:, :D].reshape(B, T, D)


# speedup vs baseline: 1.2733x; 1.0269x over previous
import functools

import jax
import jax.numpy as jnp
from jax import lax
from jax.experimental import pallas as pl
from jax.experimental.pallas import tpu as pltpu
from jax.experimental.pallas import tpu_sc as plsc

B, T, D = 64, 576, 64
K = 1024
TOK = B * T
DP = 128

BB = 2
BT = BB * T
G = TOK // BT
KC = 512
NKC = K // KC
RT = BT // 128

NC, NS = 2, 16
NW = NC * NS
CH = 128
NCHT = TOK // CH
NCH = NCHT // NW
NBUF = 3


def _argmin_body(x_ref, cb_ref, idx_ref, cbp_ref):
    halves = []
    for h in range(BB):
        xh = x_ref[h]
        best = jnp.full((1, T), jnp.inf, jnp.float32)
        besti = jnp.zeros((1, T), jnp.int32)
        iota = lax.broadcasted_iota(jnp.int32, (KC, T), 0)
        for c in range(NKC):
            cb_c = cb_ref[pl.ds(c * KC, KC), :]
            mm = lax.dot_general(cb_c, xh, (((1,), (0,)), ((), ())),
                                 preferred_element_type=jnp.float32)
            cbn = jnp.sum(cb_c * cb_c, axis=1)
            d = cbn[:, None] * 0.5 - mm
            m = jnp.min(d, axis=0, keepdims=True)
            i = jnp.min(jnp.where(d == m, iota, KC),
                        axis=0, keepdims=True)
            take = m < best
            besti = jnp.where(take, i + c * KC, besti)
            best = jnp.minimum(best, m)
        halves.append(besti)
    idx_ref[...] = jnp.concatenate(halves, axis=1).reshape(1, RT, CH)

    @pl.when(pl.program_id(0) == 0)
    def _():
        cbp_ref[...] = jnp.concatenate(
            [cb_ref[...], jnp.zeros((K, DP - D), jnp.float32)], axis=1)


_tc_argmin = pl.pallas_call(
    _argmin_body,
    grid=(G,),
    in_specs=[
        pl.BlockSpec((BB, D, T), lambda i: (i, 0, 0)),
        pl.BlockSpec((K, D), lambda i: (0, 0)),
    ],
    out_specs=[
        pl.BlockSpec((1, RT, CH), lambda i: (i, 0, 0)),
        pl.BlockSpec((K, DP), lambda i: (0, 0)),
    ],
    out_shape=[
        jax.ShapeDtypeStruct((G, RT, CH), jnp.int32),
        jax.ShapeDtypeStruct((K, DP), jnp.float32),
    ],
)


@functools.cache
def _make_sc_gather():
    @functools.partial(
        pl.kernel,
        out_type=jax.ShapeDtypeStruct((TOK, DP), jnp.float32),
        mesh=plsc.VectorSubcoreMesh(core_axis_name="c", subcore_axis_name="s"),
        scratch_types=[
            pltpu.VMEM((NCH, CH), jnp.int32),
            pltpu.VMEM((NBUF, CH, DP), jnp.float32),
            pltpu.SemaphoreType.DMA,
            pltpu.SemaphoreType.DMA,
            pltpu.SemaphoreType.DMA,
        ],
    )
    def _sc_gather(table_hbm, idx_hbm, out_hbm, idx_v, buf_v,
                   sem_i, sem_g, sem_w):
        wid = lax.axis_index("s") * NC + lax.axis_index("c")
        icp = [None] * NCH
        for t in range(NCH):
            m = wid + NW * t
            icp[t] = pltpu.async_copy(idx_hbm.at[m // RT, m % RT],
                                      idx_v.at[t], sem_i)
        gcp = [None] * NCH
        wcp = [None] * NCH
        for t in range(NCH):
            if t >= NBUF:
                wcp[t - NBUF].wait()
            icp[t].wait()
            gcp[t] = pltpu.async_copy(table_hbm.at[idx_v.at[t]],
                                      buf_v.at[t % NBUF], sem_g)
            if t >= 1:
                gcp[t - 1].wait()
                m = wid + NW * (t - 1)
                wcp[t - 1] = pltpu.async_copy(
                    buf_v.at[(t - 1) % NBUF],
                    out_hbm.at[pl.ds(m * CH, CH)], sem_w)
        gcp[NCH - 1].wait()
        m_last = wid + NW * (NCH - 1)
        wcp[NCH - 1] = pltpu.async_copy(
            buf_v.at[(NCH - 1) % NBUF],
            out_hbm.at[pl.ds(m_last * CH, CH)], sem_w)
        for t in range(NCH - NBUF, NCH):
            wcp[t].wait()

    return _sc_gather


def kernel(x, codebook):
    xt = jnp.swapaxes(x, 1, 2)
    idx, cbp = _tc_argmin(xt, codebook)
    q = _make_sc_gather()(cbp, idx)
    return q[:, :D].reshape(B, T, D)
